# baseline (speedup 1.0000x reference)
import jax
import jax.numpy as jnp
from jax import lax
from jax.experimental import pallas as pl
from jax.experimental.pallas import tpu as pltpu
from jax.experimental.pallas import tpu_sc as plsc

N_ATOMS = 1048576
N_SEG = 8192
NC = 2
NS = 16
NW = NC * NS
HALF = N_ATOMS // 2
APW = HALF // NW
CHUNK = 8192
N_CHUNKS = APW // CHUNK
GROUPS = CHUNK // 16
SEG_PER_W = N_SEG // NW
UNROLL = 8


def _vgather(x, i):
    return lax.gather(
        x, i[:, None],
        lax.GatherDimensionNumbers(
            offset_dims=(), collapsed_slice_dims=(0,), start_index_map=(0,)),
        slice_sizes=(1,),
        mode=lax.GatherScatterMode.PROMISE_IN_BOUNDS)


def _make_stage1_body(idx_off):
    def _stage1_body(x_hbm, y_hbm, z_hbm, idx_hbm, wb_hbm, part_hbm,
                     x_v, y_v, z_v, idx_v, acc_v, acc2_v, wb_v, sem0, sem1):
        wid = lax.axis_index("s") * NC + lax.axis_index("c")
        base_atom = wid * APW
        sems = (sem0, sem1)

        def start(ci):
            a0 = base_atom + ci * CHUNK
            p = ci % 2
            o = p * CHUNK
            sem = sems[p]
            return (
                pltpu.async_copy(x_hbm.at[pl.ds(a0, CHUNK)],
                                 x_v.at[pl.ds(o, CHUNK)], sem),
                pltpu.async_copy(y_hbm.at[pl.ds(a0, CHUNK)],
                                 y_v.at[pl.ds(o, CHUNK)], sem),
                pltpu.async_copy(z_hbm.at[pl.ds(a0, CHUNK)],
                                 z_v.at[pl.ds(o, CHUNK)], sem),
                pltpu.async_copy(idx_hbm.at[pl.ds(idx_off + a0, CHUNK)],
                                 idx_v.at[pl.ds(o, CHUNK)], sem),
            )

        pending = start(0)
        pltpu.sync_copy(wb_hbm, wb_v)
        wvec = wb_v[pl.ds(0, 16)]
        w0 = jnp.full((16,), wvec[0], jnp.float32)
        w1 = jnp.full((16,), wvec[1], jnp.float32)
        w2 = jnp.full((16,), wvec[2], jnp.float32)
        bb = jnp.full((16,), wvec[3], jnp.float32)
        zerosf = jnp.zeros((16,), jnp.float32)
        iota = lax.iota(jnp.int32, 16)
        nxt = jnp.minimum(iota + 1, 15)
        last_lane = iota == 15
        not_last = iota != 15

        def zbody(j, carry):
            base = j * 128
            for u in range(8):
                acc_v[pl.ds(base + u * 16, 16)] = zerosf
                acc2_v[pl.ds(base + u * 16, 16)] = zerosf
            return carry

        lax.fori_loop(0, N_SEG // 128, zbody, 0)

        for ci in range(N_CHUNKS):
            for c in pending:
                c.wait()
            if ci + 1 < N_CHUNKS:
                pending = start(ci + 1)
            pbase = (ci % 2) * CHUNK

            def g_body(g, inner, pbase=pbase):
                base = pbase + g * (16 * UNROLL)
                offs = [base + u * 16 for u in range(UNROLL)]
                xs = [x_v[pl.ds(o, 16)] for o in offs]
                ys = [y_v[pl.ds(o, 16)] for o in offs]
                zs = [z_v[pl.ds(o, 16)] for o in offs]
                ivs = [idx_v[pl.ds(o, 16)] for o in offs]
                es = [(xs[u] * w0 + ys[u] * w1) + (zs[u] * w2 + bb)
                      for u in range(UNROLL)]
                ss = [plsc.cumsum(es[u]) for u in range(UNROLL)]
                nxs = [_vgather(ivs[u], nxt) for u in range(UNROLL)]
                for u in range(UNROLL):
                    tgt = acc_v if u % 2 == 0 else acc2_v
                    nb = (ivs[u] != nxs[u]) | last_lane
                    plsc.addupdate_scatter(tgt, [ivs[u]], ss[u], mask=nb)
                    plsc.addupdate_scatter(tgt, [nxs[u]], -ss[u],
                                           mask=nb & not_last)
                return inner

            lax.fori_loop(0, GROUPS // UNROLL, g_body, 0)

        def mbody(j, carry):
            base = j * 128
            for u in range(8):
                o = base + u * 16
                acc_v[pl.ds(o, 16)] = acc_v[pl.ds(o, 16)] + acc2_v[pl.ds(o, 16)]
            return carry

        lax.fori_loop(0, N_SEG // 128, mbody, 0)
        pltpu.sync_copy(acc_v, part_hbm.at[wid])

    return _stage1_body


def _stage2_body(parta_hbm, partb_hbm, out_hbm, bufa_v, bufb_v, out_v):
    wid = lax.axis_index("s") * NC + lax.axis_index("c")
    s0 = wid * SEG_PER_W
    pltpu.sync_copy(parta_hbm.at[:, pl.ds(s0, SEG_PER_W)], bufa_v)
    pltpu.sync_copy(partb_hbm.at[:, pl.ds(s0, SEG_PER_W)], bufb_v)
    n_j = SEG_PER_W // 16

    def wbody(w, accs):
        return tuple(accs[j] + bufa_v[w, pl.ds(j * 16, 16)]
                     + bufb_v[w, pl.ds(j * 16, 16)] for j in range(n_j))

    accs = lax.fori_loop(0, NW, wbody,
                         tuple(jnp.zeros((16,), jnp.float32) for _ in range(n_j)))
    for j in range(n_j):
        out_v[pl.ds(j * 16, 16)] = accs[j]
    pltpu.sync_copy(out_v, out_hbm.at[pl.ds(s0, SEG_PER_W)])


_MESH = plsc.VectorSubcoreMesh(core_axis_name="c", subcore_axis_name="s")
_PARAMS = pltpu.CompilerParams(needs_layout_passes=False)

_STAGE1_SCRATCH = [
    pltpu.VMEM((2 * CHUNK,), jnp.float32),
    pltpu.VMEM((2 * CHUNK,), jnp.float32),
    pltpu.VMEM((2 * CHUNK,), jnp.float32),
    pltpu.VMEM((2 * CHUNK,), jnp.int32),
    pltpu.VMEM((N_SEG,), jnp.float32),
    pltpu.VMEM((N_SEG,), jnp.float32),
    pltpu.VMEM((16,), jnp.float32),
    pltpu.SemaphoreType.DMA,
    pltpu.SemaphoreType.DMA,
]

_stage1_a = pl.kernel(
    _make_stage1_body(0),
    out_type=jax.ShapeDtypeStruct((NW, N_SEG), jnp.float32),
    mesh=_MESH,
    compiler_params=_PARAMS,
    scratch_types=_STAGE1_SCRATCH,
)

_stage1_b = pl.kernel(
    _make_stage1_body(HALF),
    out_type=jax.ShapeDtypeStruct((NW, N_SEG), jnp.float32),
    mesh=_MESH,
    compiler_params=_PARAMS,
    scratch_types=_STAGE1_SCRATCH,
)

_stage2 = pl.kernel(
    _stage2_body,
    out_type=jax.ShapeDtypeStruct((N_SEG,), jnp.float32),
    mesh=_MESH,
    compiler_params=_PARAMS,
    scratch_types=[
        pltpu.VMEM((NW, SEG_PER_W), jnp.float32),
        pltpu.VMEM((NW, SEG_PER_W), jnp.float32),
        pltpu.VMEM((SEG_PER_W,), jnp.float32),
    ],
)


def kernel(positions, W, b, batch_idx):
    wb = jnp.concatenate([W.reshape(3), b.reshape(1),
                          jnp.zeros((12,), jnp.float32)])
    idx = batch_idx.astype(jnp.int32)
    pa = lax.slice(positions, (0, 0), (HALF, 3))
    pb = lax.slice(positions, (HALF, 0), (N_ATOMS, 3))
    xa = lax.slice_in_dim(pa, 0, 1, axis=1).reshape(HALF)
    ya = lax.slice_in_dim(pa, 1, 2, axis=1).reshape(HALF)
    za = lax.slice_in_dim(pa, 2, 3, axis=1).reshape(HALF)
    xb = lax.slice_in_dim(pb, 0, 1, axis=1).reshape(HALF)
    yb = lax.slice_in_dim(pb, 1, 2, axis=1).reshape(HALF)
    zb = lax.slice_in_dim(pb, 2, 3, axis=1).reshape(HALF)
    part_a = _stage1_a(xa, ya, za, idx, wb)
    part_b = _stage1_b(xb, yb, zb, idx, wb)
    energies = _stage2(part_a, part_b)
    return energies.reshape(N_SEG, 1)


# speedup vs baseline: 1.0938x; 1.0938x over previous
import jax
import jax.numpy as jnp
from jax import lax
from jax.experimental import pallas as pl
from jax.experimental.pallas import tpu as pltpu
from jax.experimental.pallas import tpu_sc as plsc

N_ATOMS = 1048576
N_SEG = 8192
NC = 2
NS = 16
NW = NC * NS
HALF = N_ATOMS // 2
APW = HALF // NW
CHUNK = 8192
N_CHUNKS = APW // CHUNK
GROUPS = CHUNK // 16
SEG_PER_W = N_SEG // NW
UNROLL = 8


def _vgather(x, i):
    return lax.gather(
        x, i[:, None],
        lax.GatherDimensionNumbers(
            offset_dims=(), collapsed_slice_dims=(0,), start_index_map=(0,)),
        slice_sizes=(1,),
        mode=lax.GatherScatterMode.PROMISE_IN_BOUNDS)


def _make_stage1_body(idx_off):
    def _stage1_body(x_hbm, y_hbm, z_hbm, idx_hbm, wb_hbm, part_hbm,
                     x_v, y_v, z_v, idx_v, acc_v, acc2_v, wb_v, sem0, sem1):
        wid = lax.axis_index("s") * NC + lax.axis_index("c")
        base_atom = wid * APW
        sems = (sem0, sem1)

        def start(ci):
            a0 = base_atom + ci * CHUNK
            p = ci % 2
            o = p * CHUNK
            sem = sems[p]
            return (
                pltpu.async_copy(x_hbm.at[pl.ds(a0, CHUNK)],
                                 x_v.at[pl.ds(o, CHUNK)], sem),
                pltpu.async_copy(y_hbm.at[pl.ds(a0, CHUNK)],
                                 y_v.at[pl.ds(o, CHUNK)], sem),
                pltpu.async_copy(z_hbm.at[pl.ds(a0, CHUNK)],
                                 z_v.at[pl.ds(o, CHUNK)], sem),
                pltpu.async_copy(idx_hbm.at[pl.ds(idx_off + a0, CHUNK)],
                                 idx_v.at[pl.ds(o, CHUNK)], sem),
            )

        pending = start(0)
        pltpu.sync_copy(wb_hbm, wb_v)
        wvec = wb_v[pl.ds(0, 16)]
        w0 = jnp.full((16,), wvec[0], jnp.float32)
        w1 = jnp.full((16,), wvec[1], jnp.float32)
        w2 = jnp.full((16,), wvec[2], jnp.float32)
        bb = jnp.full((16,), wvec[3], jnp.float32)
        zerosf = jnp.zeros((16,), jnp.float32)
        iota = lax.iota(jnp.int32, 16)
        nxt = jnp.minimum(iota + 1, 15)
        last_lane = iota == 15
        not_last = iota != 15

        def zbody(j, carry):
            base = j * 128
            for u in range(8):
                acc_v[pl.ds(base + u * 16, 16)] = zerosf
                acc2_v[pl.ds(base + u * 16, 16)] = zerosf
            return carry

        lax.fori_loop(0, N_SEG // 128, zbody, 0)

        for ci in range(N_CHUNKS):
            for c in pending:
                c.wait()
            if ci + 1 < N_CHUNKS:
                pending = start(ci + 1)
            pbase = (ci % 2) * CHUNK

            def g_body(g, inner, pbase=pbase):
                base = pbase + g * (16 * UNROLL)
                offs = [base + u * 16 for u in range(UNROLL)]
                xs = [x_v[pl.ds(o, 16)] for o in offs]
                ys = [y_v[pl.ds(o, 16)] for o in offs]
                zs = [z_v[pl.ds(o, 16)] for o in offs]
                ivs = [idx_v[pl.ds(o, 16)] for o in offs]
                es = [(xs[u] * w0 + ys[u] * w1) + (zs[u] * w2 + bb)
                      for u in range(UNROLL)]
                ss = [plsc.cumsum(es[u]) for u in range(UNROLL)]
                nxs = [_vgather(ivs[u], nxt) for u in range(UNROLL)]
                for u in range(UNROLL):
                    tgt = acc_v if u % 2 == 0 else acc2_v
                    nb = (ivs[u] != nxs[u]) | last_lane
                    plsc.addupdate_scatter(tgt, [ivs[u]], ss[u], mask=nb)
                    plsc.addupdate_scatter(tgt, [nxs[u]], -ss[u],
                                           mask=nb & not_last)
                return inner

            lax.fori_loop(0, GROUPS // UNROLL, g_body, 0)

        def mbody(j, carry):
            base = j * 128
            for u in range(8):
                o = base + u * 16
                acc_v[pl.ds(o, 16)] = acc_v[pl.ds(o, 16)] + acc2_v[pl.ds(o, 16)]
            return carry

        lax.fori_loop(0, N_SEG // 128, mbody, 0)
        pltpu.sync_copy(acc_v, part_hbm.at[wid])

    return _stage1_body


def _stage2_body(parta_hbm, partb_hbm, out_hbm, bufa_v, bufb_v, out_v):
    wid = lax.axis_index("s") * NC + lax.axis_index("c")
    s0 = wid * SEG_PER_W
    pltpu.sync_copy(parta_hbm.at[:, pl.ds(s0, SEG_PER_W)], bufa_v)
    pltpu.sync_copy(partb_hbm.at[:, pl.ds(s0, SEG_PER_W)], bufb_v)
    n_j = SEG_PER_W // 16

    def wbody(w, accs):
        return tuple(accs[j] + bufa_v[w, pl.ds(j * 16, 16)]
                     + bufb_v[w, pl.ds(j * 16, 16)] for j in range(n_j))

    accs = lax.fori_loop(0, NW, wbody,
                         tuple(jnp.zeros((16,), jnp.float32) for _ in range(n_j)))
    for j in range(n_j):
        out_v[pl.ds(j * 16, 16)] = accs[j]
    pltpu.sync_copy(out_v, out_hbm.at[pl.ds(s0, SEG_PER_W)])


_MESH = plsc.VectorSubcoreMesh(core_axis_name="c", subcore_axis_name="s")
_PARAMS = pltpu.CompilerParams(needs_layout_passes=False)

_STAGE1_SCRATCH = [
    pltpu.VMEM((2 * CHUNK,), jnp.float32),
    pltpu.VMEM((2 * CHUNK,), jnp.float32),
    pltpu.VMEM((2 * CHUNK,), jnp.float32),
    pltpu.VMEM((2 * CHUNK,), jnp.int32),
    pltpu.VMEM((N_SEG,), jnp.float32),
    pltpu.VMEM((N_SEG,), jnp.float32),
    pltpu.VMEM((16,), jnp.float32),
    pltpu.SemaphoreType.DMA,
    pltpu.SemaphoreType.DMA,
]

_stage1_a = pl.kernel(
    _make_stage1_body(0),
    out_type=jax.ShapeDtypeStruct((NW, N_SEG), jnp.float32),
    mesh=_MESH,
    compiler_params=_PARAMS,
    scratch_types=_STAGE1_SCRATCH,
)

_stage1_b = pl.kernel(
    _make_stage1_body(HALF),
    out_type=jax.ShapeDtypeStruct((NW, N_SEG), jnp.float32),
    mesh=_MESH,
    compiler_params=_PARAMS,
    scratch_types=_STAGE1_SCRATCH,
)

_stage2 = pl.kernel(
    _stage2_body,
    out_type=jax.ShapeDtypeStruct((N_SEG,), jnp.float32),
    mesh=_MESH,
    compiler_params=_PARAMS,
    scratch_types=[
        pltpu.VMEM((NW, SEG_PER_W), jnp.float32),
        pltpu.VMEM((NW, SEG_PER_W), jnp.float32),
        pltpu.VMEM((SEG_PER_W,), jnp.float32),
    ],
)


def kernel(positions, W, b, batch_idx):
    wb = jnp.concatenate([W.reshape(3), b.reshape(1),
                          jnp.zeros((12,), jnp.float32)])
    idx = batch_idx.astype(jnp.int32)
    pa = lax.slice(positions, (0, 0), (HALF, 3))
    pb = lax.slice(lax.optimization_barrier(positions), (HALF, 0), (N_ATOMS, 3))
    xa = lax.slice_in_dim(pa, 0, 1, axis=1).reshape(HALF)
    ya = lax.slice_in_dim(pa, 1, 2, axis=1).reshape(HALF)
    za = lax.slice_in_dim(pa, 2, 3, axis=1).reshape(HALF)
    xb = lax.slice_in_dim(pb, 0, 1, axis=1).reshape(HALF)
    yb = lax.slice_in_dim(pb, 1, 2, axis=1).reshape(HALF)
    zb = lax.slice_in_dim(pb, 2, 3, axis=1).reshape(HALF)
    part_a = _stage1_a(xa, ya, za, idx, wb)
    part_b = _stage1_b(xb, yb, zb, idx, wb)
    energies = _stage2(part_a, part_b)
    return energies.reshape(N_SEG, 1)


# speedup vs baseline: 1.2161x; 1.1118x over previous
import jax
import jax.numpy as jnp
from jax import lax
from jax.experimental import pallas as pl
from jax.experimental.pallas import tpu as pltpu
from jax.experimental.pallas import tpu_sc as plsc

N_ATOMS = 1048576
N_SEG = 8192
NC = 2
NS = 16
NW = NC * NS
APW = N_ATOMS // NW
CHUNK = 8192
N_CHUNKS = APW // CHUNK
BLOCKS = CHUNK // 128
SEG_PER_W = N_SEG // NW


def _vgather(x, i):
    return lax.gather(
        x, i[:, None],
        lax.GatherDimensionNumbers(
            offset_dims=(), collapsed_slice_dims=(0,), start_index_map=(0,)),
        slice_sizes=(1,),
        mode=lax.GatherScatterMode.PROMISE_IN_BOUNDS)


def _stage1_body(q_hbm, idx_hbm, wb_hbm, part_hbm,
                 q_v, idx_v, acc_v, acc2_v, wb_v, sem0, sem1):
    wid = lax.axis_index("s") * NC + lax.axis_index("c")
    base_atom = wid * APW
    sems = (sem0, sem1)

    def start(ci):
        a0 = base_atom + ci * CHUNK
        p = ci % 2
        sem = sems[p]
        return (
            pltpu.async_copy(q_hbm.at[pl.ds(a0 * 3, CHUNK * 3)],
                             q_v.at[pl.ds(p * CHUNK * 3, CHUNK * 3)], sem),
            pltpu.async_copy(idx_hbm.at[pl.ds(a0, CHUNK)],
                             idx_v.at[pl.ds(p * CHUNK, CHUNK)], sem),
        )

    pending = start(0)
    pltpu.sync_copy(wb_hbm, wb_v)
    wvec = wb_v[pl.ds(0, 16)]
    w0 = jnp.full((16,), wvec[0], jnp.float32)
    w1 = jnp.full((16,), wvec[1], jnp.float32)
    w2 = jnp.full((16,), wvec[2], jnp.float32)
    bb = jnp.full((16,), wvec[3], jnp.float32)
    zerosf = jnp.zeros((16,), jnp.float32)
    iota = lax.iota(jnp.int32, 16)
    nxt = jnp.minimum(iota + 1, 15)
    last_lane = iota == 15
    not_last = iota != 15

    def zbody(j, carry):
        base = j * 128
        for u in range(8):
            acc_v[pl.ds(base + u * 16, 16)] = zerosf
            acc2_v[pl.ds(base + u * 16, 16)] = zerosf
        return carry

    lax.fori_loop(0, N_SEG // 128, zbody, 0)

    for ci in range(N_CHUNKS):
        for c in pending:
            c.wait()
        if ci + 1 < N_CHUNKS:
            pending = start(ci + 1)
        p = ci % 2

        def blk_body(r, inner, p=p):
            qb = p * CHUNK * 3 + r * 384
            ib = p * CHUNK + r * 128
            offs = [u * 16 for u in range(8)]
            xs = [q_v[pl.ds(qb + o, 16)] for o in offs]
            ys = [q_v[pl.ds(qb + 128 + o, 16)] for o in offs]
            zs = [q_v[pl.ds(qb + 256 + o, 16)] for o in offs]
            ivs = [idx_v[pl.ds(ib + o, 16)] for o in offs]
            es = [(xs[u] * w0 + ys[u] * w1) + (zs[u] * w2 + bb)
                  for u in range(8)]
            ss = [plsc.cumsum(es[u]) for u in range(8)]
            nxs = [_vgather(ivs[u], nxt) for u in range(8)]
            for u in range(8):
                tgt = acc_v if u % 2 == 0 else acc2_v
                nb = (ivs[u] != nxs[u]) | last_lane
                plsc.addupdate_scatter(tgt, [ivs[u]], ss[u], mask=nb)
                plsc.addupdate_scatter(tgt, [nxs[u]], -ss[u],
                                       mask=nb & not_last)
            return inner

        lax.fori_loop(0, BLOCKS, blk_body, 0)

    def mbody(j, carry):
        base = j * 128
        for u in range(8):
            o = base + u * 16
            acc_v[pl.ds(o, 16)] = acc_v[pl.ds(o, 16)] + acc2_v[pl.ds(o, 16)]
        return carry

    lax.fori_loop(0, N_SEG // 128, mbody, 0)
    pltpu.sync_copy(acc_v, part_hbm.at[wid])


def _stage2_body(part_hbm, out_hbm, buf_v, out_v):
    wid = lax.axis_index("s") * NC + lax.axis_index("c")
    s0 = wid * SEG_PER_W
    pltpu.sync_copy(part_hbm.at[:, pl.ds(s0, SEG_PER_W)], buf_v)
    n_j = SEG_PER_W // 16

    def wbody(w, accs):
        return tuple(accs[j] + buf_v[w, pl.ds(j * 16, 16)] for j in range(n_j))

    accs = lax.fori_loop(0, NW, wbody,
                         tuple(jnp.zeros((16,), jnp.float32) for _ in range(n_j)))
    for j in range(n_j):
        out_v[pl.ds(j * 16, 16)] = accs[j]
    pltpu.sync_copy(out_v, out_hbm.at[pl.ds(s0, SEG_PER_W)])


_MESH = plsc.VectorSubcoreMesh(core_axis_name="c", subcore_axis_name="s")
_PARAMS = pltpu.CompilerParams(needs_layout_passes=False)

_stage1 = pl.kernel(
    _stage1_body,
    out_type=jax.ShapeDtypeStruct((NW, N_SEG), jnp.float32),
    mesh=_MESH,
    compiler_params=_PARAMS,
    scratch_types=[
        pltpu.VMEM((2 * CHUNK * 3,), jnp.float32),
        pltpu.VMEM((2 * CHUNK,), jnp.int32),
        pltpu.VMEM((N_SEG,), jnp.float32),
        pltpu.VMEM((N_SEG,), jnp.float32),
        pltpu.VMEM((16,), jnp.float32),
        pltpu.SemaphoreType.DMA,
        pltpu.SemaphoreType.DMA,
    ],
)

_stage2 = pl.kernel(
    _stage2_body,
    out_type=jax.ShapeDtypeStruct((N_SEG,), jnp.float32),
    mesh=_MESH,
    compiler_params=_PARAMS,
    scratch_types=[
        pltpu.VMEM((NW, SEG_PER_W), jnp.float32),
        pltpu.VMEM((SEG_PER_W,), jnp.float32),
    ],
)


def kernel(positions, W, b, batch_idx):
    wb = jnp.concatenate([W.reshape(3), b.reshape(1),
                          jnp.zeros((12,), jnp.float32)])
    idx = batch_idx.astype(jnp.int32)
    q = (positions.T.reshape(3, N_ATOMS // 128, 128)
         .transpose(1, 0, 2).reshape(N_ATOMS * 3))
    part = _stage1(q, idx, wb)
    energies = _stage2(part)
    return energies.reshape(N_SEG, 1)


# speedup vs baseline: 1.2164x; 1.0002x over previous
import jax
import jax.numpy as jnp
from jax import lax
from jax.experimental import pallas as pl
from jax.experimental.pallas import tpu as pltpu
from jax.experimental.pallas import tpu_sc as plsc

N_ATOMS = 1048576
N_SEG = 8192
NC = 2
NS = 16
NW = NC * NS
APW = N_ATOMS // NW
CHUNK = 8192
N_CHUNKS = APW // CHUNK
BLOCKS = CHUNK // 128
SEG_PER_W = N_SEG // NW


def _vgather(x, i):
    return lax.gather(
        x, i[:, None],
        lax.GatherDimensionNumbers(
            offset_dims=(), collapsed_slice_dims=(0,), start_index_map=(0,)),
        slice_sizes=(1,),
        mode=lax.GatherScatterMode.PROMISE_IN_BOUNDS)


def _stage1_body(q_hbm, idx_hbm, wb_hbm, part_hbm,
                 q_v, idx_v, acc_v, acc2_v, wb_v, sem0, sem1):
    wid = lax.axis_index("s") * NC + lax.axis_index("c")
    base_atom = wid * APW
    sems = (sem0, sem1)

    def start(ci):
        a0 = base_atom + ci * CHUNK
        p = ci % 2
        sem = sems[p]
        return (
            pltpu.async_copy(q_hbm.at[pl.ds(a0 * 3, CHUNK * 3)],
                             q_v.at[pl.ds(p * CHUNK * 3, CHUNK * 3)], sem),
            pltpu.async_copy(idx_hbm.at[pl.ds(a0, CHUNK)],
                             idx_v.at[pl.ds(p * CHUNK, CHUNK)], sem),
        )

    pending = start(0)
    pltpu.sync_copy(wb_hbm, wb_v)
    wvec = wb_v[pl.ds(0, 16)]
    w0 = jnp.full((16,), wvec[0], jnp.float32)
    w1 = jnp.full((16,), wvec[1], jnp.float32)
    w2 = jnp.full((16,), wvec[2], jnp.float32)
    bb = jnp.full((16,), wvec[3], jnp.float32)
    zerosf = jnp.zeros((16,), jnp.float32)
    iota = lax.iota(jnp.int32, 16)
    nxt = jnp.minimum(iota + 1, 15)
    last_lane = iota == 15
    not_last = iota != 15

    def zbody(j, carry):
        base = j * 128
        for u in range(8):
            acc_v[pl.ds(base + u * 16, 16)] = zerosf
            acc2_v[pl.ds(base + u * 16, 16)] = zerosf
        return carry

    lax.fori_loop(0, N_SEG // 128, zbody, 0)

    for ci in range(N_CHUNKS):
        for c in pending:
            c.wait()
        if ci + 1 < N_CHUNKS:
            pending = start(ci + 1)
        p = ci % 2

        def blk_body(r, inner, p=p):
            qb = p * CHUNK * 3 + r * 384
            ib = p * CHUNK + r * 128
            offs = [u * 16 for u in range(8)]
            xs = [q_v[pl.ds(qb + o, 16)] for o in offs]
            ys = [q_v[pl.ds(qb + 128 + o, 16)] for o in offs]
            zs = [q_v[pl.ds(qb + 256 + o, 16)] for o in offs]
            ivs = [idx_v[pl.ds(ib + o, 16)] for o in offs]
            es = [(xs[u] * w0 + ys[u] * w1) + (zs[u] * w2 + bb)
                  for u in range(8)]
            ss = [plsc.cumsum(es[u]) for u in range(8)]
            nxs = [_vgather(ivs[u], nxt) for u in range(8)]
            for u in range(8):
                tgt = acc_v if u % 2 == 0 else acc2_v
                nb = (ivs[u] != nxs[u]) | last_lane
                plsc.addupdate_scatter(tgt, [ivs[u]], ss[u], mask=nb)
                plsc.addupdate_scatter(acc2_v if u % 2 == 0 else acc_v,
                                       [nxs[u]], -ss[u],
                                       mask=nb & not_last)
            return inner

        lax.fori_loop(0, BLOCKS, blk_body, 0)

    def mbody(j, carry):
        base = j * 128
        for u in range(8):
            o = base + u * 16
            acc_v[pl.ds(o, 16)] = acc_v[pl.ds(o, 16)] + acc2_v[pl.ds(o, 16)]
        return carry

    lax.fori_loop(0, N_SEG // 128, mbody, 0)
    pltpu.sync_copy(acc_v, part_hbm.at[wid])


def _stage2_body(part_hbm, out_hbm, buf_v, out_v):
    wid = lax.axis_index("s") * NC + lax.axis_index("c")
    s0 = wid * SEG_PER_W
    pltpu.sync_copy(part_hbm.at[:, pl.ds(s0, SEG_PER_W)], buf_v)
    n_j = SEG_PER_W // 16

    def wbody(w, accs):
        return tuple(accs[j] + buf_v[w, pl.ds(j * 16, 16)] for j in range(n_j))

    accs = lax.fori_loop(0, NW, wbody,
                         tuple(jnp.zeros((16,), jnp.float32) for _ in range(n_j)))
    for j in range(n_j):
        out_v[pl.ds(j * 16, 16)] = accs[j]
    pltpu.sync_copy(out_v, out_hbm.at[pl.ds(s0, SEG_PER_W)])


_MESH = plsc.VectorSubcoreMesh(core_axis_name="c", subcore_axis_name="s")
_PARAMS = pltpu.CompilerParams(needs_layout_passes=False)

_stage1 = pl.kernel(
    _stage1_body,
    out_type=jax.ShapeDtypeStruct((NW, N_SEG), jnp.float32),
    mesh=_MESH,
    compiler_params=_PARAMS,
    scratch_types=[
        pltpu.VMEM((2 * CHUNK * 3,), jnp.float32),
        pltpu.VMEM((2 * CHUNK,), jnp.int32),
        pltpu.VMEM((N_SEG,), jnp.float32),
        pltpu.VMEM((N_SEG,), jnp.float32),
        pltpu.VMEM((16,), jnp.float32),
        pltpu.SemaphoreType.DMA,
        pltpu.SemaphoreType.DMA,
    ],
)

_stage2 = pl.kernel(
    _stage2_body,
    out_type=jax.ShapeDtypeStruct((N_SEG,), jnp.float32),
    mesh=_MESH,
    compiler_params=_PARAMS,
    scratch_types=[
        pltpu.VMEM((NW, SEG_PER_W), jnp.float32),
        pltpu.VMEM((SEG_PER_W,), jnp.float32),
    ],
)


def kernel(positions, W, b, batch_idx):
    wb = jnp.concatenate([W.reshape(3), b.reshape(1),
                          jnp.zeros((12,), jnp.float32)])
    idx = batch_idx.astype(jnp.int32)
    q = (positions.T.reshape(3, N_ATOMS // 128, 128)
         .transpose(1, 0, 2).reshape(N_ATOMS * 3))
    part = _stage1(q, idx, wb)
    energies = _stage2(part)
    return energies.reshape(N_SEG, 1)
